# baseline (speedup 1.0000x reference)
import jax
import jax.numpy as jnp
from jax import lax
from jax.experimental import pallas as pl
from jax.experimental.pallas import tpu as pltpu
from jax.experimental.pallas import tpu_sc as plsc

D = 64
DB = 2 * D
C = 128
NBUF = 2
NC = 2
NS = 16
NW = NC * NS
E_PAD = 1 << 20
W_EDGES = E_PAD // NW
CPW = W_EDGES // C
N_IDX_ROWS = E_PAD // C


def _dot_kernel(tab_hbm, idx_hbm, out_hbm,
                idxu_v, idxm_v, rows_u, rows_m, out_a,
                sem_u, sem_m, sem_out):
  wid = lax.axis_index("s") * NC + lax.axis_index("c")
  base_w = wid * W_EDGES
  row_w = wid * CPW
  lane = lax.iota(jnp.int32, 16)
  hi_mask = jnp.full((16,), -65536, jnp.int32)

  pltpu.sync_copy(idx_hbm.at[pl.ds(row_w, CPW)], idxu_v)
  pltpu.sync_copy(idx_hbm.at[pl.ds(N_IDX_ROWS + row_w, CPW)], idxm_v)

  def issue(t, b):
    pltpu.async_copy(tab_hbm.at[idxu_v.at[t]], rows_u[b], sem_u[b])
    pltpu.async_copy(tab_hbm.at[idxm_v.at[t]], rows_m[b], sem_m[b])

  def wait(t, b):
    pltpu.make_async_copy(tab_hbm.at[idxu_v.at[t]], rows_u[b], sem_u[b]).wait()
    pltpu.make_async_copy(tab_hbm.at[idxm_v.at[t]], rows_m[b], sem_m[b]).wait()

  def split_f32(v32):
    vi = plsc.bitcast(v32, jnp.int32)
    lo = plsc.bitcast(vi << 16, jnp.float32)
    hi = plsc.bitcast(vi & hi_mask, jnp.float32)
    return lo, hi

  def compute(b, half):
    ru = rows_u[b]
    rm = rows_m[b]
    oa = out_a[half]

    @pl.loop(0, C // 16)
    def _group(g):
      res = jnp.zeros((16,), jnp.float32)
      for l in range(16):
        e = g * 16 + l
        acc = None
        for k in range(D // 32):
          ulo, uhi = split_f32(ru[e, pl.ds(k * 32, 32)])
          mlo, mhi = split_f32(rm[e, pl.ds(k * 32, 32)])
          term = ulo * mlo + uhi * mhi
          acc = term if acc is None else acc + term
        s = jnp.sum(acc)
        res = jnp.where(lane == l, s, res)
      oa[pl.ds(b * C + g * 16, 16)] = res

  for b in range(NBUF):
    issue(b, b)

  @pl.loop(0, CPW, step=2 * NBUF)
  def _superstep(c):
    for half in range(2):
      t0 = c + half * NBUF
      @pl.when(c > 0)
      def _():
        pltpu.make_async_copy(
            out_a[half], out_hbm.at[pl.ds(0, NBUF * C)], sem_out[half]).wait()
      for b in range(NBUF):
        t = t0 + b
        wait(t, b)
        compute(b, half)
        nxt = t + NBUF
        @pl.when(nxt < CPW)
        def _():
          issue(nxt, b)
      pltpu.async_copy(
          out_a[half], out_hbm.at[pl.ds(base_w + t0 * C, NBUF * C)],
          sem_out[half])

  for half in range(2):
    pltpu.make_async_copy(
        out_a[half], out_hbm.at[pl.ds(0, NBUF * C)], sem_out[half]).wait()


@jax.jit
def kernel(x_user, x_movie, edge_label_index):
  n_edges = edge_label_index.shape[1]
  n_rows = x_user.shape[0]
  iu = edge_label_index[0].astype(jnp.int32)
  im = edge_label_index[1].astype(jnp.int32) + n_rows
  pad = E_PAD - n_edges
  zer = jnp.zeros((pad,), jnp.int32)
  idx = jnp.concatenate([iu, zer, im, zer]).reshape(2 * N_IDX_ROWS, C)
  tab = jnp.concatenate(
      [x_user.astype(jnp.bfloat16), x_movie.astype(jnp.bfloat16)])

  mesh = plsc.VectorSubcoreMesh(core_axis_name="c", subcore_axis_name="s")
  run = pl.kernel(
      _dot_kernel,
      out_type=jax.ShapeDtypeStruct((E_PAD,), jnp.float32),
      mesh=mesh,
      scratch_types=[
          pltpu.VMEM((CPW, C), jnp.int32),
          pltpu.VMEM((CPW, C), jnp.int32),
          [pltpu.VMEM((C, D), jnp.bfloat16) for _ in range(NBUF)],
          [pltpu.VMEM((C, D), jnp.bfloat16) for _ in range(NBUF)],
          [pltpu.VMEM((NBUF * C,), jnp.float32) for _ in range(2)],
          [pltpu.SemaphoreType.DMA for _ in range(NBUF)],
          [pltpu.SemaphoreType.DMA for _ in range(NBUF)],
          [pltpu.SemaphoreType.DMA for _ in range(2)],
      ],
      compiler_params=pltpu.CompilerParams(
          needs_layout_passes=False, use_tc_tiling_on_sc=False),
  )
  out = run(tab, idx)
  return out[:n_edges]


# speedup vs baseline: 1.7668x; 1.7668x over previous
import jax
import jax.numpy as jnp
from jax import lax
from jax.experimental import pallas as pl
from jax.experimental.pallas import tpu as pltpu
from jax.experimental.pallas import tpu_sc as plsc

D = 64
DB = 2 * D
C = 128
NBUF = 2
NC = 2
NS = 16
NW = NC * NS
E_PAD = 1 << 20
W_EDGES = E_PAD // NW
CPW = W_EDGES // C
N_IDX_ROWS = E_PAD // C


def _dot_kernel(xu_hbm, xm_hbm, iu_hbm, im_hbm, out_hbm,
                idxu_v, idxm_v, rows_u, rows_m, out_a,
                sem_u, sem_m, sem_out):
  wid = lax.axis_index("s") * NC + lax.axis_index("c")
  base_w = wid * W_EDGES
  row_w = wid * CPW
  lane = lax.iota(jnp.int32, 16)
  hi_mask = jnp.full((16,), -65536, jnp.int32)

  pltpu.sync_copy(iu_hbm.at[pl.ds(row_w, CPW)], idxu_v)
  pltpu.sync_copy(im_hbm.at[pl.ds(row_w, CPW)], idxm_v)

  def issue(t, b):
    pltpu.async_copy(xu_hbm.at[idxu_v.at[t]], rows_u[b], sem_u[b])
    pltpu.async_copy(xm_hbm.at[idxm_v.at[t]], rows_m[b], sem_m[b])

  def wait(t, b):
    pltpu.make_async_copy(xu_hbm.at[idxu_v.at[t]], rows_u[b], sem_u[b]).wait()
    pltpu.make_async_copy(xm_hbm.at[idxm_v.at[t]], rows_m[b], sem_m[b]).wait()

  def split_f32(v32):
    vi = plsc.bitcast(v32, jnp.int32)
    lo = plsc.bitcast(vi << 16, jnp.float32)
    hi = plsc.bitcast(vi & hi_mask, jnp.float32)
    return lo, hi

  def compute(b, half):
    ru = rows_u[b]
    rm = rows_m[b]
    oa = out_a[half]

    @pl.loop(0, C // 16)
    def _group(g):
      res = jnp.zeros((16,), jnp.float32)
      for l in range(16):
        e = g * 16 + l
        acc = None
        for k in range(D // 32):
          ulo, uhi = split_f32(ru[e, pl.ds(k * 32, 32)])
          mlo, mhi = split_f32(rm[e, pl.ds(k * 32, 32)])
          term = ulo * mlo + uhi * mhi
          acc = term if acc is None else acc + term
        s = jnp.sum(acc)
        res = jnp.where(lane == l, s, res)
      oa[pl.ds(b * C + g * 16, 16)] = res

  for b in range(NBUF):
    issue(b, b)

  @pl.loop(0, CPW, step=2 * NBUF)
  def _superstep(c):
    for half in range(2):
      t0 = c + half * NBUF
      @pl.when(c > 0)
      def _():
        pltpu.make_async_copy(
            out_a[half], out_hbm.at[pl.ds(0, NBUF * C)], sem_out[half]).wait()
      for b in range(NBUF):
        t = t0 + b
        wait(t, b)
        compute(b, half)
        nxt = t + NBUF
        @pl.when(nxt < CPW)
        def _():
          issue(nxt, b)
      pltpu.async_copy(
          out_a[half], out_hbm.at[pl.ds(base_w + t0 * C, NBUF * C)],
          sem_out[half])

  for half in range(2):
    pltpu.make_async_copy(
        out_a[half], out_hbm.at[pl.ds(0, NBUF * C)], sem_out[half]).wait()


@jax.jit
def kernel(x_user, x_movie, edge_label_index):
  n_edges = edge_label_index.shape[1]
  iu = edge_label_index[0].astype(jnp.int32)
  im = edge_label_index[1].astype(jnp.int32)
  pad = E_PAD - n_edges
  iu = jnp.concatenate([iu, jnp.zeros((pad,), jnp.int32)]).reshape(N_IDX_ROWS, C)
  im = jnp.concatenate([im, jnp.zeros((pad,), jnp.int32)]).reshape(N_IDX_ROWS, C)

  mesh = plsc.VectorSubcoreMesh(core_axis_name="c", subcore_axis_name="s")
  run = pl.kernel(
      _dot_kernel,
      out_type=jax.ShapeDtypeStruct((E_PAD,), jnp.float32),
      mesh=mesh,
      scratch_types=[
          pltpu.VMEM((CPW, C), jnp.int32),
          pltpu.VMEM((CPW, C), jnp.int32),
          [pltpu.VMEM((C, D), jnp.bfloat16) for _ in range(NBUF)],
          [pltpu.VMEM((C, D), jnp.bfloat16) for _ in range(NBUF)],
          [pltpu.VMEM((NBUF * C,), jnp.float32) for _ in range(2)],
          [pltpu.SemaphoreType.DMA for _ in range(NBUF)],
          [pltpu.SemaphoreType.DMA for _ in range(NBUF)],
          [pltpu.SemaphoreType.DMA for _ in range(2)],
      ],
      compiler_params=pltpu.CompilerParams(
          needs_layout_passes=False, use_tc_tiling_on_sc=False),
  )
  out = run(x_user.astype(jnp.bfloat16), x_movie.astype(jnp.bfloat16), iu, im)
  return out[:n_edges]
